# baseline (speedup 1.0000x reference)
import functools

import jax
import jax.numpy as jnp
from jax import lax
from jax.experimental import pallas as pl
from jax.experimental.pallas import tpu as pltpu
from jax.experimental.pallas import tpu_sc as plsc

VOCAB = 1000000
D = 64
B = 16384
NEG = 20
NC = 2
NS = 16
NW = NC * NS
NB = B // NW
SUB = 16
NSUB = NB // SUB
IDXW = 80
NIR = SUB * NEG // IDXW
NROW = SUB * NEG
CW = 2 * D

TRC = 2048
TRG = (VOCAB + TRC - 1) // TRC


def _tr_body(ut_ref, vt_ref, o_ref):
    r = lax.broadcasted_iota(jnp.int32, (D, D), 0)
    c = lax.broadcasted_iota(jnp.int32, (D, D), 1)
    ident = (r == c).astype(jnp.float32)
    dn = (((0,), (0,)), ((), ()))
    o_ref[:, pl.ds(0, D)] = lax.dot_general(
        ut_ref[...], ident, dn, preferred_element_type=jnp.float32)
    o_ref[:, pl.ds(D, D)] = lax.dot_general(
        vt_ref[...], ident, dn, preferred_element_type=jnp.float32)


_tr_call = pl.pallas_call(
    _tr_body,
    grid=(TRG,),
    in_specs=[pl.BlockSpec((D, TRC), lambda i: (0, i)),
              pl.BlockSpec((D, TRC), lambda i: (0, i))],
    out_specs=pl.BlockSpec((TRC, CW), lambda i: (i, 0)),
    out_shape=jax.ShapeDtypeStruct((VOCAB, CW), jnp.float32),
)


def _sc_body(uidx_h, vidx_h, nidx_h, C_h, score_h, negsc_h,
             uidx_v, vidx_v, nidx_v, ubuf, vbuf, nbuf, pbuf, qbuf,
             sstage, qstage, sem0, sem1):
    cid = lax.axis_index("c")
    sid = lax.axis_index("s")
    wid = sid * NC + cid
    r0 = wid * (NB // 128)

    pltpu.sync_copy(uidx_h.at[pl.ds(wid * NSUB, NSUB)], uidx_v)
    pltpu.sync_copy(vidx_h.at[pl.ds(wid * NSUB, NSUB)], vidx_v)
    pltpu.sync_copy(nidx_h.at[pl.ds(wid * 128, 128)], nidx_v)

    sems = (sem0, sem1)

    def issue(s, slot):
        sem = sems[slot]
        pltpu.async_copy(C_h.at[uidx_v.at[s]], ubuf.at[slot], sem)
        pltpu.async_copy(C_h.at[vidx_v.at[s]], vbuf.at[slot], sem)
        for j in range(NIR):
            pltpu.async_copy(C_h.at[nidx_v.at[s * NIR + j]],
                             nbuf.at[slot].at[pl.ds(j * IDXW, IDXW)], sem)

    def drain(s, slot):
        sem = sems[slot]
        pltpu.make_async_copy(C_h.at[uidx_v.at[s]], ubuf.at[slot], sem).wait()
        pltpu.make_async_copy(C_h.at[vidx_v.at[s]], vbuf.at[slot], sem).wait()
        for j in range(NIR):
            pltpu.make_async_copy(C_h.at[nidx_v.at[s * NIR + j]],
                                  nbuf.at[slot].at[pl.ds(j * IDXW, IDXW)],
                                  sem).wait()

    issue(0, 0)

    rows16 = lax.iota(jnp.int32, 16)

    def compute(s, slot):
        nslot = nbuf.at[slot]
        ub = ubuf.at[slot]
        vb = vbuf.at[slot]

        def bbody(i, carry):
            u = [ub[i, pl.ds(16 * k, 16)] for k in range(4)]
            v = [vb[i, pl.ds(D + 16 * k, 16)] for k in range(4)]
            p = u[0] * v[0] + u[1] * v[1] + u[2] * v[2] + u[3] * v[3]
            base = i * NEG
            acc = [nslot[base, pl.ds(D + 16 * k, 16)] for k in range(4)]
            for n in range(1, NEG):
                for k in range(4):
                    acc[k] = acc[k] + nslot[base + n, pl.ds(D + 16 * k, 16)]
            q = (u[0] * acc[0] + u[1] * acc[1]
                 + u[2] * acc[2] + u[3] * acc[3])
            pbuf[i] = p
            qbuf[i] = q
            return carry

        lax.fori_loop(0, SUB, bbody, 0)

        sc = jnp.zeros((16,), jnp.float32)
        qc = jnp.zeros((16,), jnp.float32)
        for k in range(16):
            kk = jnp.full((16,), k, jnp.int32)
            sc = sc + plsc.load_gather(pbuf, [rows16, kk])
            qc = qc + plsc.load_gather(qbuf, [rows16, kk])
        b0 = s * SUB
        rr = b0 // 128
        cc = b0 % 128
        sstage[rr, pl.ds(cc, 16)] = sc
        qstage[rr, pl.ds(cc, 16)] = qc

    def step(t, carry):
        s_even = 2 * t
        issue(s_even + 1, 1)
        drain(s_even, 0)
        compute(s_even, 0)

        @pl.when(s_even + 2 < NSUB)
        def _():
            issue(s_even + 2, 0)

        drain(s_even + 1, 1)
        compute(s_even + 1, 1)
        return carry

    lax.fori_loop(0, NSUB // 2, step, 0)

    pltpu.sync_copy(sstage, score_h.at[pl.ds(r0, NB // 128)])
    pltpu.sync_copy(qstage, negsc_h.at[pl.ds(r0, NB // 128)])


@functools.cache
def _sc_call_cached():
    return functools.partial(
        pl.kernel,
        out_type=(jax.ShapeDtypeStruct((B // 128, 128), jnp.float32),
                  jax.ShapeDtypeStruct((B // 128, 128), jnp.float32)),
        mesh=plsc.VectorSubcoreMesh(core_axis_name="c", subcore_axis_name="s",
                                    num_cores=NC, num_subcores=NS),
        compiler_params=pltpu.CompilerParams(needs_layout_passes=False,
                                             use_tc_tiling_on_sc=False),
        scratch_types=[
            pltpu.VMEM((NSUB, SUB), jnp.int32),
            pltpu.VMEM((NSUB, SUB), jnp.int32),
            pltpu.VMEM((128, IDXW), jnp.int32),
            pltpu.VMEM((2, SUB, CW), jnp.float32),
            pltpu.VMEM((2, SUB, CW), jnp.float32),
            pltpu.VMEM((2, NROW, CW), jnp.float32),
            pltpu.VMEM((16, 16), jnp.float32),
            pltpu.VMEM((16, 16), jnp.float32),
            pltpu.VMEM((NB // 128, 128), jnp.float32),
            pltpu.VMEM((NB // 128, 128), jnp.float32),
            pltpu.SemaphoreType.DMA,
            pltpu.SemaphoreType.DMA,
        ],
    )(_sc_body)


def _loss_body(s_ref, q_ref, o_ref):
    s = s_ref[...]
    q = q_ref[...]
    ls = jnp.minimum(s, 0.0) - jnp.log(1.0 + jnp.exp(-jnp.abs(s)))
    lq = jnp.minimum(-q, 0.0) - jnp.log(1.0 + jnp.exp(-jnp.abs(q)))
    o_ref[0, 0] = -(jnp.sum(ls) + jnp.sum(lq)) / jnp.float32(B)


_loss_call = pl.pallas_call(
    _loss_body,
    out_shape=jax.ShapeDtypeStruct((1, 1), jnp.float32),
    out_specs=pl.BlockSpec(memory_space=pltpu.SMEM),
)


def kernel(u_idx, v_idx, v_neg, U, V):
    u2 = u_idx.astype(jnp.int32).reshape(B // SUB, SUB)
    v2 = v_idx.astype(jnp.int32).reshape(B // SUB, SUB)
    n2 = v_neg.astype(jnp.int32).reshape(B * NEG // IDXW, IDXW)
    comb = _tr_call(U.T, V.T)
    score, negsc = _sc_call_cached()(u2, v2, n2, comb)
    out = _loss_call(score, negsc)
    return out[0, 0]


# speedup vs baseline: 1.0006x; 1.0006x over previous
import functools

import jax
import jax.numpy as jnp
from jax import lax
from jax.experimental import pallas as pl
from jax.experimental.pallas import tpu as pltpu
from jax.experimental.pallas import tpu_sc as plsc

VOCAB = 1000000
D = 64
B = 16384
NEG = 20
NC = 2
NS = 16
NW = NC * NS
NB = B // NW
SUB = 16
NSUB = NB // SUB
IDXW = 80
NIR = SUB * NEG // IDXW
NROW = SUB * NEG
CW = 2 * D

TRC = 2048
TRG = (VOCAB + TRC - 1) // TRC


def _tr_body(ut_ref, vt_ref, o_ref):
    o_ref[:, pl.ds(0, D)] = ut_ref[...].T
    o_ref[:, pl.ds(D, D)] = vt_ref[...].T


_tr_call = pl.pallas_call(
    _tr_body,
    grid=(TRG,),
    in_specs=[pl.BlockSpec((D, TRC), lambda i: (0, i)),
              pl.BlockSpec((D, TRC), lambda i: (0, i))],
    out_specs=pl.BlockSpec((TRC, CW), lambda i: (i, 0)),
    out_shape=jax.ShapeDtypeStruct((VOCAB, CW), jnp.float32),
)


def _sc_body(uidx_h, vidx_h, nidx_h, C_h, score_h, negsc_h,
             uidx_v, vidx_v, nidx_v, ubuf, vbuf, nbuf, pbuf, qbuf,
             sstage, qstage, sem0, sem1):
    cid = lax.axis_index("c")
    sid = lax.axis_index("s")
    wid = sid * NC + cid
    r0 = wid * (NB // 128)

    pltpu.sync_copy(uidx_h.at[pl.ds(wid * NSUB, NSUB)], uidx_v)
    pltpu.sync_copy(vidx_h.at[pl.ds(wid * NSUB, NSUB)], vidx_v)
    pltpu.sync_copy(nidx_h.at[pl.ds(wid * 128, 128)], nidx_v)

    sems = (sem0, sem1)

    def issue(s, slot):
        sem = sems[slot]
        pltpu.async_copy(C_h.at[uidx_v.at[s]], ubuf.at[slot], sem)
        pltpu.async_copy(C_h.at[vidx_v.at[s]], vbuf.at[slot], sem)
        for j in range(NIR):
            pltpu.async_copy(C_h.at[nidx_v.at[s * NIR + j]],
                             nbuf.at[slot].at[pl.ds(j * IDXW, IDXW)], sem)

    def drain(s, slot):
        sem = sems[slot]
        pltpu.make_async_copy(C_h.at[uidx_v.at[s]], ubuf.at[slot], sem).wait()
        pltpu.make_async_copy(C_h.at[vidx_v.at[s]], vbuf.at[slot], sem).wait()
        for j in range(NIR):
            pltpu.make_async_copy(C_h.at[nidx_v.at[s * NIR + j]],
                                  nbuf.at[slot].at[pl.ds(j * IDXW, IDXW)],
                                  sem).wait()

    issue(0, 0)

    rows16 = lax.iota(jnp.int32, 16)

    def compute(s, slot):
        nslot = nbuf.at[slot]
        ub = ubuf.at[slot]
        vb = vbuf.at[slot]

        def bbody(i, carry):
            u = [ub[i, pl.ds(16 * k, 16)] for k in range(4)]
            v = [vb[i, pl.ds(D + 16 * k, 16)] for k in range(4)]
            p = u[0] * v[0] + u[1] * v[1] + u[2] * v[2] + u[3] * v[3]
            base = i * NEG
            acc = [nslot[base, pl.ds(D + 16 * k, 16)] for k in range(4)]
            for n in range(1, NEG):
                for k in range(4):
                    acc[k] = acc[k] + nslot[base + n, pl.ds(D + 16 * k, 16)]
            q = (u[0] * acc[0] + u[1] * acc[1]
                 + u[2] * acc[2] + u[3] * acc[3])
            pbuf[i] = p
            qbuf[i] = q
            return carry

        lax.fori_loop(0, SUB, bbody, 0)

        sc = jnp.zeros((16,), jnp.float32)
        qc = jnp.zeros((16,), jnp.float32)
        for k in range(16):
            kk = jnp.full((16,), k, jnp.int32)
            sc = sc + plsc.load_gather(pbuf, [rows16, kk])
            qc = qc + plsc.load_gather(qbuf, [rows16, kk])
        b0 = s * SUB
        rr = b0 // 128
        cc = b0 % 128
        sstage[rr, pl.ds(cc, 16)] = sc
        qstage[rr, pl.ds(cc, 16)] = qc

    def step(t, carry):
        s_even = 2 * t
        issue(s_even + 1, 1)
        drain(s_even, 0)
        compute(s_even, 0)

        @pl.when(s_even + 2 < NSUB)
        def _():
            issue(s_even + 2, 0)

        drain(s_even + 1, 1)
        compute(s_even + 1, 1)
        return carry

    lax.fori_loop(0, NSUB // 2, step, 0)

    pltpu.sync_copy(sstage, score_h.at[pl.ds(r0, NB // 128)])
    pltpu.sync_copy(qstage, negsc_h.at[pl.ds(r0, NB // 128)])


@functools.cache
def _sc_call_cached():
    return functools.partial(
        pl.kernel,
        out_type=(jax.ShapeDtypeStruct((B // 128, 128), jnp.float32),
                  jax.ShapeDtypeStruct((B // 128, 128), jnp.float32)),
        mesh=plsc.VectorSubcoreMesh(core_axis_name="c", subcore_axis_name="s",
                                    num_cores=NC, num_subcores=NS),
        compiler_params=pltpu.CompilerParams(needs_layout_passes=False,
                                             use_tc_tiling_on_sc=False),
        scratch_types=[
            pltpu.VMEM((NSUB, SUB), jnp.int32),
            pltpu.VMEM((NSUB, SUB), jnp.int32),
            pltpu.VMEM((128, IDXW), jnp.int32),
            pltpu.VMEM((2, SUB, CW), jnp.float32),
            pltpu.VMEM((2, SUB, CW), jnp.float32),
            pltpu.VMEM((2, NROW, CW), jnp.float32),
            pltpu.VMEM((16, 16), jnp.float32),
            pltpu.VMEM((16, 16), jnp.float32),
            pltpu.VMEM((NB // 128, 128), jnp.float32),
            pltpu.VMEM((NB // 128, 128), jnp.float32),
            pltpu.SemaphoreType.DMA,
            pltpu.SemaphoreType.DMA,
        ],
    )(_sc_body)


def _loss_body(s_ref, q_ref, o_ref):
    s = s_ref[...]
    q = q_ref[...]
    ls = jnp.minimum(s, 0.0) - jnp.log(1.0 + jnp.exp(-jnp.abs(s)))
    lq = jnp.minimum(-q, 0.0) - jnp.log(1.0 + jnp.exp(-jnp.abs(q)))
    o_ref[0, 0] = -(jnp.sum(ls) + jnp.sum(lq)) / jnp.float32(B)


_loss_call = pl.pallas_call(
    _loss_body,
    out_shape=jax.ShapeDtypeStruct((1, 1), jnp.float32),
    out_specs=pl.BlockSpec(memory_space=pltpu.SMEM),
)


def kernel(u_idx, v_idx, v_neg, U, V):
    u2 = u_idx.astype(jnp.int32).reshape(B // SUB, SUB)
    v2 = v_idx.astype(jnp.int32).reshape(B // SUB, SUB)
    n2 = v_neg.astype(jnp.int32).reshape(B * NEG // IDXW, IDXW)
    comb = _tr_call(U.T, V.T)
    score, negsc = _sc_call_cached()(u2, v2, n2, comb)
    out = _loss_call(score, negsc)
    return out[0, 0]


# speedup vs baseline: 1.3664x; 1.3656x over previous
import functools

import jax
import jax.numpy as jnp
from jax import lax
from jax.experimental import pallas as pl
from jax.experimental.pallas import tpu as pltpu
from jax.experimental.pallas import tpu_sc as plsc

VOCAB = 1000000
D = 64
B = 16384
NEG = 20
NC = 2
NS = 16
NW = NC * NS
NB = B // NW
SUB = 16
NSUB = NB // SUB
IDXW = 80
NIR = SUB * NEG // IDXW
NROW = SUB * NEG
CW = 2 * D

TRC = 8192
TRG = (VOCAB + TRC - 1) // TRC


def _tr_body(ut_ref, vt_ref, o_ref):
    o_ref[:, pl.ds(0, D)] = ut_ref[...].T
    o_ref[:, pl.ds(D, D)] = vt_ref[...].T


_tr_call = pl.pallas_call(
    _tr_body,
    grid=(TRG,),
    in_specs=[pl.BlockSpec((D, TRC), lambda i: (0, i)),
              pl.BlockSpec((D, TRC), lambda i: (0, i))],
    out_specs=pl.BlockSpec((TRC, CW), lambda i: (i, 0)),
    out_shape=jax.ShapeDtypeStruct((VOCAB, CW), jnp.float32),
)


def _sc_body(uidx_h, vidx_h, nidx_h, C_h, score_h, negsc_h,
             uidx_v, vidx_v, nidx_v, ubuf, vbuf, nbuf, pbuf, qbuf,
             sstage, qstage, sem0, sem1):
    cid = lax.axis_index("c")
    sid = lax.axis_index("s")
    wid = sid * NC + cid
    r0 = wid * (NB // 128)

    pltpu.sync_copy(uidx_h.at[pl.ds(wid * NSUB, NSUB)], uidx_v)
    pltpu.sync_copy(vidx_h.at[pl.ds(wid * NSUB, NSUB)], vidx_v)
    pltpu.sync_copy(nidx_h.at[pl.ds(wid * 128, 128)], nidx_v)

    sems = (sem0, sem1)

    def issue(s, slot):
        sem = sems[slot]
        pltpu.async_copy(C_h.at[uidx_v.at[s]], ubuf.at[slot], sem)
        pltpu.async_copy(C_h.at[vidx_v.at[s]], vbuf.at[slot], sem)
        for j in range(NIR):
            pltpu.async_copy(C_h.at[nidx_v.at[s * NIR + j]],
                             nbuf.at[slot].at[pl.ds(j * IDXW, IDXW)], sem)

    def drain(s, slot):
        sem = sems[slot]
        pltpu.make_async_copy(C_h.at[uidx_v.at[s]], ubuf.at[slot], sem).wait()
        pltpu.make_async_copy(C_h.at[vidx_v.at[s]], vbuf.at[slot], sem).wait()
        for j in range(NIR):
            pltpu.make_async_copy(C_h.at[nidx_v.at[s * NIR + j]],
                                  nbuf.at[slot].at[pl.ds(j * IDXW, IDXW)],
                                  sem).wait()

    issue(0, 0)

    rows16 = lax.iota(jnp.int32, 16)

    def compute(s, slot):
        nslot = nbuf.at[slot]
        ub = ubuf.at[slot]
        vb = vbuf.at[slot]

        def bbody(i, carry):
            u = [ub[i, pl.ds(16 * k, 16)] for k in range(4)]
            v = [vb[i, pl.ds(D + 16 * k, 16)] for k in range(4)]
            p = u[0] * v[0] + u[1] * v[1] + u[2] * v[2] + u[3] * v[3]
            base = i * NEG
            acc = [nslot[base, pl.ds(D + 16 * k, 16)] for k in range(4)]
            for n in range(1, NEG):
                for k in range(4):
                    acc[k] = acc[k] + nslot[base + n, pl.ds(D + 16 * k, 16)]
            q = (u[0] * acc[0] + u[1] * acc[1]
                 + u[2] * acc[2] + u[3] * acc[3])
            pbuf[i] = p
            qbuf[i] = q
            return carry

        lax.fori_loop(0, SUB, bbody, 0)

        sc = jnp.zeros((16,), jnp.float32)
        qc = jnp.zeros((16,), jnp.float32)
        for k in range(16):
            kk = jnp.full((16,), k, jnp.int32)
            sc = sc + plsc.load_gather(pbuf, [rows16, kk])
            qc = qc + plsc.load_gather(qbuf, [rows16, kk])
        b0 = s * SUB
        rr = b0 // 128
        cc = b0 % 128
        sstage[rr, pl.ds(cc, 16)] = sc
        qstage[rr, pl.ds(cc, 16)] = qc

    def step(t, carry):
        s_even = 2 * t
        issue(s_even + 1, 1)
        drain(s_even, 0)
        compute(s_even, 0)

        @pl.when(s_even + 2 < NSUB)
        def _():
            issue(s_even + 2, 0)

        drain(s_even + 1, 1)
        compute(s_even + 1, 1)
        return carry

    lax.fori_loop(0, NSUB // 2, step, 0)

    pltpu.sync_copy(sstage, score_h.at[pl.ds(r0, NB // 128)])
    pltpu.sync_copy(qstage, negsc_h.at[pl.ds(r0, NB // 128)])


@functools.cache
def _sc_call_cached():
    return functools.partial(
        pl.kernel,
        out_type=(jax.ShapeDtypeStruct((B // 128, 128), jnp.float32),
                  jax.ShapeDtypeStruct((B // 128, 128), jnp.float32)),
        mesh=plsc.VectorSubcoreMesh(core_axis_name="c", subcore_axis_name="s",
                                    num_cores=NC, num_subcores=NS),
        compiler_params=pltpu.CompilerParams(needs_layout_passes=False,
                                             use_tc_tiling_on_sc=False),
        scratch_types=[
            pltpu.VMEM((NSUB, SUB), jnp.int32),
            pltpu.VMEM((NSUB, SUB), jnp.int32),
            pltpu.VMEM((128, IDXW), jnp.int32),
            pltpu.VMEM((2, SUB, CW), jnp.float32),
            pltpu.VMEM((2, SUB, CW), jnp.float32),
            pltpu.VMEM((2, NROW, CW), jnp.float32),
            pltpu.VMEM((16, 16), jnp.float32),
            pltpu.VMEM((16, 16), jnp.float32),
            pltpu.VMEM((NB // 128, 128), jnp.float32),
            pltpu.VMEM((NB // 128, 128), jnp.float32),
            pltpu.SemaphoreType.DMA,
            pltpu.SemaphoreType.DMA,
        ],
    )(_sc_body)


def _loss_body(s_ref, q_ref, o_ref):
    s = s_ref[...]
    q = q_ref[...]
    ls = jnp.minimum(s, 0.0) - jnp.log(1.0 + jnp.exp(-jnp.abs(s)))
    lq = jnp.minimum(-q, 0.0) - jnp.log(1.0 + jnp.exp(-jnp.abs(q)))
    o_ref[0, 0] = -(jnp.sum(ls) + jnp.sum(lq)) / jnp.float32(B)


_loss_call = pl.pallas_call(
    _loss_body,
    out_shape=jax.ShapeDtypeStruct((1, 1), jnp.float32),
    out_specs=pl.BlockSpec(memory_space=pltpu.SMEM),
)


def kernel(u_idx, v_idx, v_neg, U, V):
    u2 = u_idx.astype(jnp.int32).reshape(B // SUB, SUB)
    v2 = v_idx.astype(jnp.int32).reshape(B // SUB, SUB)
    n2 = v_neg.astype(jnp.int32).reshape(B * NEG // IDXW, IDXW)
    comb = _tr_call(U.T, V.T)
    score, negsc = _sc_call_cached()(u2, v2, n2, comb)
    out = _loss_call(score, negsc)
    return out[0, 0]


# speedup vs baseline: 1.4480x; 1.0597x over previous
import functools

import jax
import jax.numpy as jnp
from jax import lax
from jax.experimental import pallas as pl
from jax.experimental.pallas import tpu as pltpu
from jax.experimental.pallas import tpu_sc as plsc

VOCAB = 1000000
D = 64
B = 16384
NEG = 20
NC = 2
NS = 16
NW = NC * NS
NB = B // NW
SUB = 16
NSUB = NB // SUB
IDXW = 80
NIR = SUB * NEG // IDXW
NROW = SUB * NEG
CW = 2 * D

TRC = 16384
TRG = (VOCAB + TRC - 1) // TRC


def _tr_body(ut_ref, vt_ref, o_ref):
    o_ref[:, pl.ds(0, D)] = ut_ref[...].T
    o_ref[:, pl.ds(D, D)] = vt_ref[...].T


_tr_call = pl.pallas_call(
    _tr_body,
    grid=(TRG,),
    in_specs=[pl.BlockSpec((D, TRC), lambda i: (0, i)),
              pl.BlockSpec((D, TRC), lambda i: (0, i))],
    out_specs=pl.BlockSpec((TRC, CW), lambda i: (i, 0)),
    out_shape=jax.ShapeDtypeStruct((VOCAB, CW), jnp.float32),
)


def _sc_body(uidx_h, vidx_h, nidx_h, C_h, score_h, negsc_h,
             uidx_v, vidx_v, nidx_v, ubuf, vbuf, nbuf, pbuf, qbuf,
             sstage, qstage, sem0, sem1):
    cid = lax.axis_index("c")
    sid = lax.axis_index("s")
    wid = sid * NC + cid
    r0 = wid * (NB // 128)

    pltpu.sync_copy(uidx_h.at[pl.ds(wid * NSUB, NSUB)], uidx_v)
    pltpu.sync_copy(vidx_h.at[pl.ds(wid * NSUB, NSUB)], vidx_v)
    pltpu.sync_copy(nidx_h.at[pl.ds(wid * 128, 128)], nidx_v)

    sems = (sem0, sem1)

    def issue(s, slot):
        sem = sems[slot]
        pltpu.async_copy(C_h.at[uidx_v.at[s]], ubuf.at[slot], sem)
        pltpu.async_copy(C_h.at[vidx_v.at[s]], vbuf.at[slot], sem)
        for j in range(NIR):
            pltpu.async_copy(C_h.at[nidx_v.at[s * NIR + j]],
                             nbuf.at[slot].at[pl.ds(j * IDXW, IDXW)], sem)

    def drain(s, slot):
        sem = sems[slot]
        pltpu.make_async_copy(C_h.at[uidx_v.at[s]], ubuf.at[slot], sem).wait()
        pltpu.make_async_copy(C_h.at[vidx_v.at[s]], vbuf.at[slot], sem).wait()
        for j in range(NIR):
            pltpu.make_async_copy(C_h.at[nidx_v.at[s * NIR + j]],
                                  nbuf.at[slot].at[pl.ds(j * IDXW, IDXW)],
                                  sem).wait()

    issue(0, 0)

    rows16 = lax.iota(jnp.int32, 16)

    def compute(s, slot):
        nslot = nbuf.at[slot]
        ub = ubuf.at[slot]
        vb = vbuf.at[slot]

        def bbody(i, carry):
            u = [ub[i, pl.ds(16 * k, 16)] for k in range(4)]
            v = [vb[i, pl.ds(D + 16 * k, 16)] for k in range(4)]
            p = u[0] * v[0] + u[1] * v[1] + u[2] * v[2] + u[3] * v[3]
            base = i * NEG
            acc = [nslot[base, pl.ds(D + 16 * k, 16)] for k in range(4)]
            for n in range(1, NEG):
                for k in range(4):
                    acc[k] = acc[k] + nslot[base + n, pl.ds(D + 16 * k, 16)]
            q = (u[0] * acc[0] + u[1] * acc[1]
                 + u[2] * acc[2] + u[3] * acc[3])
            pbuf[i] = p
            qbuf[i] = q
            return carry

        lax.fori_loop(0, SUB, bbody, 0)

        sc = jnp.zeros((16,), jnp.float32)
        qc = jnp.zeros((16,), jnp.float32)
        for k in range(16):
            kk = jnp.full((16,), k, jnp.int32)
            sc = sc + plsc.load_gather(pbuf, [rows16, kk])
            qc = qc + plsc.load_gather(qbuf, [rows16, kk])
        b0 = s * SUB
        rr = b0 // 128
        cc = b0 % 128
        sstage[rr, pl.ds(cc, 16)] = sc
        qstage[rr, pl.ds(cc, 16)] = qc

    def step(t, carry):
        s_even = 2 * t
        issue(s_even + 1, 1)
        drain(s_even, 0)
        compute(s_even, 0)

        @pl.when(s_even + 2 < NSUB)
        def _():
            issue(s_even + 2, 0)

        drain(s_even + 1, 1)
        compute(s_even + 1, 1)
        return carry

    lax.fori_loop(0, NSUB // 2, step, 0)

    pltpu.sync_copy(sstage, score_h.at[pl.ds(r0, NB // 128)])
    pltpu.sync_copy(qstage, negsc_h.at[pl.ds(r0, NB // 128)])


@functools.cache
def _sc_call_cached():
    return functools.partial(
        pl.kernel,
        out_type=(jax.ShapeDtypeStruct((B // 128, 128), jnp.float32),
                  jax.ShapeDtypeStruct((B // 128, 128), jnp.float32)),
        mesh=plsc.VectorSubcoreMesh(core_axis_name="c", subcore_axis_name="s",
                                    num_cores=NC, num_subcores=NS),
        compiler_params=pltpu.CompilerParams(needs_layout_passes=False,
                                             use_tc_tiling_on_sc=False),
        scratch_types=[
            pltpu.VMEM((NSUB, SUB), jnp.int32),
            pltpu.VMEM((NSUB, SUB), jnp.int32),
            pltpu.VMEM((128, IDXW), jnp.int32),
            pltpu.VMEM((2, SUB, CW), jnp.float32),
            pltpu.VMEM((2, SUB, CW), jnp.float32),
            pltpu.VMEM((2, NROW, CW), jnp.float32),
            pltpu.VMEM((16, 16), jnp.float32),
            pltpu.VMEM((16, 16), jnp.float32),
            pltpu.VMEM((NB // 128, 128), jnp.float32),
            pltpu.VMEM((NB // 128, 128), jnp.float32),
            pltpu.SemaphoreType.DMA,
            pltpu.SemaphoreType.DMA,
        ],
    )(_sc_body)


def _loss_body(s_ref, q_ref, o_ref):
    s = s_ref[...]
    q = q_ref[...]
    ls = jnp.minimum(s, 0.0) - jnp.log(1.0 + jnp.exp(-jnp.abs(s)))
    lq = jnp.minimum(-q, 0.0) - jnp.log(1.0 + jnp.exp(-jnp.abs(q)))
    o_ref[0, 0] = -(jnp.sum(ls) + jnp.sum(lq)) / jnp.float32(B)


_loss_call = pl.pallas_call(
    _loss_body,
    out_shape=jax.ShapeDtypeStruct((1, 1), jnp.float32),
    out_specs=pl.BlockSpec(memory_space=pltpu.SMEM),
)


def kernel(u_idx, v_idx, v_neg, U, V):
    u2 = u_idx.astype(jnp.int32).reshape(B // SUB, SUB)
    v2 = v_idx.astype(jnp.int32).reshape(B // SUB, SUB)
    n2 = v_neg.astype(jnp.int32).reshape(B * NEG // IDXW, IDXW)
    comb = _tr_call(U.T, V.T)
    score, negsc = _sc_call_cached()(u2, v2, n2, comb)
    out = _loss_call(score, negsc)
    return out[0, 0]
